# Initial kernel scaffold; baseline (speedup 1.0000x reference)
import jax
import jax.numpy as jnp
from jax.experimental import pallas as pl


def kernel(x, edge_index, batch, W1, b1, W2, b2, Wl1, bl1, Wl2, bl2):
    raise NotImplementedError("write your pallas kernel here")


# speedup vs baseline: 3.9557x; 3.9557x over previous
import jax
import jax.numpy as jnp
from jax import lax
from jax.experimental import pallas as pl
from jax.experimental.pallas import tpu as pltpu
from jax.experimental.pallas import tpu_sc as plsc

N = 10000
E = 320000
D = 128
H = 128
B = 64
C = 10

NC = 2
NT = 16
HH = H // 2
K = 128
NBUF = 2
EPT = -(-E // (NT * K * 8)) * (K * 8)
NCHT = EPT // K
EPAD = EPT * NT
NWD = NC * NT
EPWD = EPAD // NWD
NCHD = EPWD // K
NP = 10240
RPT = NP // NT
HN = 5120
HNP = 6144
RPH = HNP // NT
DUMP = HN
NPX = 10496
RCH = 256
DW = 16

_mesh = plsc.VectorSubcoreMesh(core_axis_name="c", subcore_axis_name="s",
                               num_cores=NC, num_subcores=NT)
_mesh1 = plsc.VectorSubcoreMesh(core_axis_name="c", subcore_axis_name="s",
                                num_cores=1, num_subcores=NT)


def _deg_body(dst_hbm, out_hbm, dst_v, obuf, acc_sh):
    sid = lax.axis_index("s")
    pltpu.sync_copy(dst_hbm.at[pl.ds(sid * NCHT, NCHT)], dst_v)
    zeros16 = jnp.zeros((16,), jnp.float32)
    ones16 = jnp.ones((16,), jnp.float32)

    @pl.loop(0, K)
    def _fill_zero(r):
        for cc in range(H // 16):
            obuf[r, pl.ds(cc * 16, 16)] = zeros16

    @pl.loop(0, RPT // K)
    def _zero_acc(z):
        pltpu.sync_copy(obuf, acc_sh.at[pl.ds(sid * RPT + z * K, K)])

    plsc.subcore_barrier()

    @pl.loop(0, K)
    def _fill_one(r):
        for cc in range(H // 16):
            obuf[r, pl.ds(cc * 16, 16)] = ones16

    @pl.loop(0, NCHT)
    def _scatter(j):
        pltpu.sync_copy(obuf, acc_sh.at[dst_v.at[j]], add=True)

    plsc.subcore_barrier()
    pltpu.sync_copy(acc_sh.at[pl.ds(sid * RPT, RPT)],
                    out_hbm.at[pl.ds(sid * RPT, RPT)])


_deg = pl.kernel(
    _deg_body,
    out_type=jax.ShapeDtypeStruct((NP, H), jnp.float32),
    mesh=_mesh1,
    scratch_types=[
        pltpu.VMEM((NCHT, K), jnp.int32),
        pltpu.VMEM((K, H), jnp.float32),
        pltpu.VMEM_SHARED((NP, H), jnp.float32),
    ],
)


def _edge_agg_body(ys_hbm, src_hbm, dst_hbm, out_hbm,
                   src_v, dst_v, dst_r, r0, r1, acc_sh, s0, s1):
    rows = (r0, r1)
    sems = (s0, s1)
    sid = lax.axis_index("s")
    pltpu.sync_copy(src_hbm.at[pl.ds(sid * NCHT, NCHT)], src_v)
    pltpu.sync_copy(dst_hbm.at[pl.ds(sid * NCHT, NCHT)], dst_v)
    zeros16 = jnp.zeros((16,), jnp.float32)

    for h in range(2):
        @pl.loop(0, K)
        def _zero_rows(r):
            for cc in range(H // 16):
                r0[r, pl.ds(cc * 16, 16)] = zeros16

        @pl.loop(0, RPH // K)
        def _zero_acc(z):
            pltpu.sync_copy(r0, acc_sh.at[pl.ds(sid * RPH + z * K, K)])

        plsc.subcore_barrier()

        for b in range(NBUF):
            pltpu.async_copy(ys_hbm.at[src_v.at[b]], rows[b], sems[b])

        @pl.loop(0, NCHT, step=NBUF)
        def _edges(j0):
            for b in range(NBUF):
                j = j0 + b
                for g in range(K // 16):
                    d = dst_v[j, pl.ds(g * 16, 16)] - (h * HN)
                    ok = (d >= 0) & (d < HN)
                    dst_r[b, pl.ds(g * 16, 16)] = jnp.where(ok, d, DUMP)
                pltpu.make_async_copy(ys_hbm.at[src_v.at[j]], rows[b],
                                      sems[b]).wait()
                pltpu.sync_copy(rows[b], acc_sh.at[dst_r.at[b]], add=True)

                @pl.when(j + NBUF < NCHT)
                def _():
                    pltpu.async_copy(ys_hbm.at[src_v.at[j + NBUF]], rows[b],
                                     sems[b])

        plsc.subcore_barrier()
        pltpu.sync_copy(acc_sh.at[pl.ds(sid * RPH, RPH)],
                        out_hbm.at[pl.ds(h * HNP + sid * RPH, RPH)])


_edge_agg = pl.kernel(
    _edge_agg_body,
    out_type=jax.ShapeDtypeStruct((2 * HNP, H), jnp.float32),
    mesh=_mesh1,
    scratch_types=[
        pltpu.VMEM((NCHT, K), jnp.int32),
        pltpu.VMEM((NCHT, K), jnp.int32),
        pltpu.VMEM((NBUF, K), jnp.int32),
        pltpu.VMEM((K, H), jnp.float32),
        pltpu.VMEM((K, H), jnp.float32),
        pltpu.VMEM_SHARED((HNP, H), jnp.float32),
        pltpu.SemaphoreType.DMA,
        pltpu.SemaphoreType.DMA,
    ],
)


def _pool_into(xs_ref, batch_row, mean_ref, max_ref):
    def seg_body(b, carry):
        lt = jnp.sum((batch_row < b).astype(jnp.int32))
        ln = jnp.sum((batch_row == b).astype(jnp.int32))
        nch = lax.div(ln + (RCH - 1), RCH)

        def ch_body(c, sc):
            sm, mx = sc
            base = lt + c * RCH
            chunk = xs_ref[pl.ds(base, RCH), :]
            rid = lax.broadcasted_iota(jnp.int32, (RCH, 1), 0) + base
            msk = rid < lt + ln
            sm = sm + jnp.sum(jnp.where(msk, chunk, 0.0), axis=0, keepdims=True)
            mx = jnp.maximum(mx, jnp.max(jnp.where(msk, chunk, -jnp.inf),
                                         axis=0, keepdims=True))
            return sm, mx

        sm, mx = lax.fori_loop(
            0, nch, ch_body,
            (jnp.zeros((1, H), jnp.float32),
             jnp.full((1, H), -jnp.inf, jnp.float32)))
        mean_ref[pl.ds(b, 1), :] = sm / jnp.maximum(ln.astype(jnp.float32), 1.0)
        max_ref[pl.ds(b, 1), :] = mx
        return carry

    lax.fori_loop(0, B, seg_body, 0)


def _tc_a_body(degp_ref, x_ref, w1_ref, y1s_ref, dinv_ref):
    deg = degp_ref[0:N, 0:1] + 1.0
    dinv = lax.rsqrt(jnp.maximum(deg, 1.0))
    xw = jnp.dot(x_ref[...], w1_ref[...], preferred_element_type=jnp.float32)
    y1s_ref[...] = xw * dinv
    dinv_ref[...] = dinv


_tc_a = pl.pallas_call(
    _tc_a_body,
    out_shape=(jax.ShapeDtypeStruct((N, H), jnp.float32),
               jax.ShapeDtypeStruct((N, 1), jnp.float32)),
)


def _tc_b_body(pp_ref, y1s_ref, dinv_ref, b1_ref, batch_ref, w2_ref,
               y2s_ref, g0m_ref, g0x_ref, xs_ref):
    es = jnp.concatenate([pp_ref[0:HN, :],
                          pp_ref[HNP:HNP + (N - HN), :]], axis=0)
    acc = es + y1s_ref[...]
    x0 = jnp.maximum(acc * dinv_ref[...] + b1_ref[...], 0.0)
    xs_ref[:N, :] = x0
    y2s_ref[...] = jnp.dot(x0, w2_ref[...],
                           preferred_element_type=jnp.float32) * dinv_ref[...]
    _pool_into(xs_ref, batch_ref[...], g0m_ref, g0x_ref)


_tc_b = pl.pallas_call(
    _tc_b_body,
    out_shape=(jax.ShapeDtypeStruct((N, H), jnp.float32),
               jax.ShapeDtypeStruct((B, H), jnp.float32),
               jax.ShapeDtypeStruct((B, H), jnp.float32)),
    scratch_shapes=[pltpu.VMEM((NPX, H), jnp.float32)],
)


def _tc_d_body(pp_ref, y2s_ref, dinv_ref, b2_ref, batch_ref,
               g0m_ref, g0x_ref, wl1_ref, bl1_ref, wl2_ref, bl2_ref,
               out_ref, xs_ref, g1m_ref, g1x_ref):
    es = jnp.concatenate([pp_ref[0:HN, :],
                          pp_ref[HNP:HNP + (N - HN), :]], axis=0)
    acc = es + y2s_ref[...]
    x1 = jnp.maximum(acc * dinv_ref[...] + b2_ref[...], 0.0)
    xs_ref[:N, :] = x1
    _pool_into(xs_ref, batch_ref[...], g1m_ref, g1x_ref)
    g = jnp.concatenate([g0m_ref[...], g0x_ref[...],
                         g1m_ref[...], g1x_ref[...]], axis=1)
    h = jnp.maximum(jnp.dot(g, wl1_ref[...],
                            preferred_element_type=jnp.float32) + bl1_ref[...],
                    0.0)
    out_ref[...] = jnp.dot(h, wl2_ref[...],
                           preferred_element_type=jnp.float32) + bl2_ref[...]


_tc_d = pl.pallas_call(
    _tc_d_body,
    out_shape=jax.ShapeDtypeStruct((B, C), jnp.float32),
    scratch_shapes=[
        pltpu.VMEM((NPX, H), jnp.float32),
        pltpu.VMEM((B, H), jnp.float32),
        pltpu.VMEM((B, H), jnp.float32),
    ],
)


def kernel(x, edge_index, batch, W1, b1, W2, b2, Wl1, bl1, Wl2, bl2):
    ei = edge_index.astype(jnp.int32)
    src = ei[0]
    dst = ei[1]
    pad = EPAD - E
    src_p = jnp.concatenate([src, jnp.zeros((pad,), jnp.int32)])
    dst_p = jnp.concatenate([dst, jnp.full((pad,), N, jnp.int32)])
    src_w = src_p.reshape(NT * NCHT, K)
    dst_w = dst_p.reshape(NT * NCHT, K)
    batch_row = batch.astype(jnp.int32).reshape(1, N)

    degp = _deg(dst_w)
    y1s, dinv = _tc_a(degp, x, W1)
    pp1 = _edge_agg(y1s, src_w, dst_w)
    y2s, g0m, g0x = _tc_b(pp1, y1s, dinv, b1.reshape(1, H), batch_row, W2)
    pp2 = _edge_agg(y2s, src_w, dst_w)
    out = _tc_d(pp2, y2s, dinv, b2.reshape(1, H), batch_row, g0m, g0x,
                Wl1, bl1.reshape(1, 2 * H), Wl2, bl2.reshape(1, C))
    return out


# speedup vs baseline: 7.8496x; 1.9844x over previous
import jax
import jax.numpy as jnp
from jax import lax
from jax.experimental import pallas as pl
from jax.experimental.pallas import tpu as pltpu
from jax.experimental.pallas import tpu_sc as plsc

N = 10000
E = 320000
D = 128
H = 128
B = 64
C = 10

NC = 2
NT = 16
HH = H // 2
K = 128
NBUF = 2
EPT = -(-E // (NT * K * 8)) * (K * 8)
NCHT = EPT // K
EPAD = EPT * NT
NWD = NC * NT
EPWD = EPAD // NWD
NCHD = EPWD // K
NP = 10240
RPT = NP // NT
SB = 8
NBLK = 160 // SB
NPX = 10496
RCH = 256
DW = 16

_mesh = plsc.VectorSubcoreMesh(core_axis_name="c", subcore_axis_name="s",
                               num_cores=NC, num_subcores=NT)
_mesh1 = plsc.VectorSubcoreMesh(core_axis_name="c", subcore_axis_name="s",
                                num_cores=1, num_subcores=NT)


def _deg_body(dst_hbm, out_hbm, dst_v, obuf, acc_sh):
    sid = lax.axis_index("s")
    pltpu.sync_copy(dst_hbm.at[pl.ds(sid * NCHT, NCHT)], dst_v)
    zeros16 = jnp.zeros((16,), jnp.float32)
    ones16 = jnp.ones((16,), jnp.float32)

    @pl.loop(0, K)
    def _fill_zero(r):
        for cc in range(H // 16):
            obuf[r, pl.ds(cc * 16, 16)] = zeros16

    @pl.loop(0, RPT // K)
    def _zero_acc(z):
        pltpu.sync_copy(obuf, acc_sh.at[pl.ds(sid * RPT + z * K, K)])

    plsc.subcore_barrier()

    @pl.loop(0, K)
    def _fill_one(r):
        for cc in range(H // 16):
            obuf[r, pl.ds(cc * 16, 16)] = ones16

    @pl.loop(0, NCHT)
    def _scatter(j):
        pltpu.sync_copy(obuf, acc_sh.at[dst_v.at[j]], add=True)

    plsc.subcore_barrier()
    pltpu.sync_copy(acc_sh.at[pl.ds(sid * RPT, RPT)],
                    out_hbm.at[pl.ds(sid * RPT, RPT)])


_deg = pl.kernel(
    _deg_body,
    out_type=jax.ShapeDtypeStruct((NP, H), jnp.float32),
    mesh=_mesh1,
    scratch_types=[
        pltpu.VMEM((NCHT, K), jnp.int32),
        pltpu.VMEM((K, H), jnp.float32),
        pltpu.VMEM_SHARED((NP, H), jnp.float32),
    ],
)


def _edge_agg_body(ys_hbm, src_hbm, dst_hbm, out_hbm,
                   src_s, dst_s, r0, r1, acc_sh, g0, g1, t0, t1):
    rows = (r0, r1)
    gsems = (g0, g1)
    tsems = (t0, t1)
    sid = lax.axis_index("s")
    base = sid * NCHT

    def stage(blk, slot, sem):
        pltpu.async_copy(src_hbm.at[pl.ds(base + blk * SB, SB)],
                         src_s.at[slot], sem)
        pltpu.async_copy(dst_hbm.at[pl.ds(base + blk * SB, SB)],
                         dst_s.at[slot], sem)

    def stage_wait(slot, sem):
        pltpu.make_async_copy(src_hbm.at[pl.ds(0, SB)],
                              src_s.at[slot], sem).wait()
        pltpu.make_async_copy(dst_hbm.at[pl.ds(0, SB)],
                              dst_s.at[slot], sem).wait()

    zeros16 = jnp.zeros((16,), jnp.float32)

    @pl.loop(0, K)
    def _zero_rows(r):
        for cc in range(H // 16):
            r0[r, pl.ds(cc * 16, 16)] = zeros16

    @pl.loop(0, RPT // K)
    def _zero_acc(z):
        pltpu.sync_copy(r0, acc_sh.at[pl.ds(sid * RPT + z * K, K)])

    plsc.subcore_barrier()

    stage(0, 0, tsems[0])
    stage_wait(0, tsems[0])
    pltpu.async_copy(ys_hbm.at[src_s.at[0, 0]], rows[0], gsems[0])
    pltpu.async_copy(ys_hbm.at[src_s.at[0, 1]], rows[1], gsems[1])

    @pl.loop(0, NBLK, step=2)
    def _blocks(bi0):
        for sb in range(2):
            bi = bi0 + sb
            oslot = 1 - sb

            @pl.when(bi + 1 < NBLK)
            def _():
                stage(bi + 1, oslot, tsems[oslot])

            for cc in range(SB):
                j = bi * SB + cc
                b = cc % 2
                pltpu.make_async_copy(ys_hbm.at[src_s.at[sb, cc]], rows[b],
                                      gsems[b]).wait()
                pltpu.sync_copy(rows[b], acc_sh.at[dst_s.at[sb, cc]],
                                add=True)
                if cc == SB - 2:
                    @pl.when(bi + 1 < NBLK)
                    def _():
                        stage_wait(oslot, tsems[oslot])
                jj = j + 2
                cc2 = cc + 2
                if cc2 < SB:
                    @pl.when(jj < NCHT)
                    def _():
                        pltpu.async_copy(ys_hbm.at[src_s.at[sb, cc2]],
                                         rows[b], gsems[b])
                else:
                    @pl.when(jj < NCHT)
                    def _():
                        pltpu.async_copy(ys_hbm.at[src_s.at[oslot, cc2 - SB]],
                                         rows[b], gsems[b])

    plsc.subcore_barrier()
    pltpu.sync_copy(acc_sh.at[pl.ds(sid * RPT, RPT)],
                    out_hbm.at[pl.ds(sid * RPT, RPT)])


_edge_agg = pl.kernel(
    _edge_agg_body,
    out_type=jax.ShapeDtypeStruct((NP, H), jnp.float32),
    mesh=_mesh1,
    scratch_types=[
        pltpu.VMEM((2, SB, K), jnp.int32),
        pltpu.VMEM((2, SB, K), jnp.int32),
        pltpu.VMEM((K, H), jnp.float32),
        pltpu.VMEM((K, H), jnp.float32),
        pltpu.VMEM_SHARED((NP, H), jnp.float32),
        pltpu.SemaphoreType.DMA,
        pltpu.SemaphoreType.DMA,
        pltpu.SemaphoreType.DMA,
        pltpu.SemaphoreType.DMA,
    ],
)


def _pool_into(xs_ref, batch_row, mean_ref, max_ref):
    def seg_body(b, carry):
        lt = jnp.sum((batch_row < b).astype(jnp.int32))
        ln = jnp.sum((batch_row == b).astype(jnp.int32))
        nch = lax.div(ln + (RCH - 1), RCH)

        def ch_body(c, sc):
            sm, mx = sc
            base = lt + c * RCH
            chunk = xs_ref[pl.ds(base, RCH), :]
            rid = lax.broadcasted_iota(jnp.int32, (RCH, 1), 0) + base
            msk = rid < lt + ln
            sm = sm + jnp.sum(jnp.where(msk, chunk, 0.0), axis=0, keepdims=True)
            mx = jnp.maximum(mx, jnp.max(jnp.where(msk, chunk, -jnp.inf),
                                         axis=0, keepdims=True))
            return sm, mx

        sm, mx = lax.fori_loop(
            0, nch, ch_body,
            (jnp.zeros((1, H), jnp.float32),
             jnp.full((1, H), -jnp.inf, jnp.float32)))
        mean_ref[pl.ds(b, 1), :] = sm / jnp.maximum(ln.astype(jnp.float32), 1.0)
        max_ref[pl.ds(b, 1), :] = mx
        return carry

    lax.fori_loop(0, B, seg_body, 0)


def _tc_a_body(degp_ref, x_ref, w1_ref, y1s_ref, dinv_ref):
    deg = degp_ref[0:N, 0:1] + 1.0
    dinv = lax.rsqrt(jnp.maximum(deg, 1.0))
    xw = jnp.dot(x_ref[...], w1_ref[...], preferred_element_type=jnp.float32)
    y1s_ref[...] = xw * dinv
    dinv_ref[...] = dinv


_tc_a = pl.pallas_call(
    _tc_a_body,
    out_shape=(jax.ShapeDtypeStruct((N, H), jnp.float32),
               jax.ShapeDtypeStruct((N, 1), jnp.float32)),
)


def _tc_b_body(pp_ref, y1s_ref, dinv_ref, b1_ref, batch_ref, w2_ref,
               y2s_ref, g0m_ref, g0x_ref, xs_ref):
    acc = pp_ref[0:N, :] + y1s_ref[...]
    x0 = jnp.maximum(acc * dinv_ref[...] + b1_ref[...], 0.0)
    xs_ref[:N, :] = x0
    y2s_ref[...] = jnp.dot(x0, w2_ref[...],
                           preferred_element_type=jnp.float32) * dinv_ref[...]
    _pool_into(xs_ref, batch_ref[...], g0m_ref, g0x_ref)


_tc_b = pl.pallas_call(
    _tc_b_body,
    out_shape=(jax.ShapeDtypeStruct((N, H), jnp.float32),
               jax.ShapeDtypeStruct((B, H), jnp.float32),
               jax.ShapeDtypeStruct((B, H), jnp.float32)),
    scratch_shapes=[pltpu.VMEM((NPX, H), jnp.float32)],
)


def _tc_d_body(pp_ref, y2s_ref, dinv_ref, b2_ref, batch_ref,
               g0m_ref, g0x_ref, wl1_ref, bl1_ref, wl2_ref, bl2_ref,
               out_ref, xs_ref, g1m_ref, g1x_ref):
    acc = pp_ref[0:N, :] + y2s_ref[...]
    x1 = jnp.maximum(acc * dinv_ref[...] + b2_ref[...], 0.0)
    xs_ref[:N, :] = x1
    _pool_into(xs_ref, batch_ref[...], g1m_ref, g1x_ref)
    g = jnp.concatenate([g0m_ref[...], g0x_ref[...],
                         g1m_ref[...], g1x_ref[...]], axis=1)
    h = jnp.maximum(jnp.dot(g, wl1_ref[...],
                            preferred_element_type=jnp.float32) + bl1_ref[...],
                    0.0)
    out_ref[...] = jnp.dot(h, wl2_ref[...],
                           preferred_element_type=jnp.float32) + bl2_ref[...]


_tc_d = pl.pallas_call(
    _tc_d_body,
    out_shape=jax.ShapeDtypeStruct((B, C), jnp.float32),
    scratch_shapes=[
        pltpu.VMEM((NPX, H), jnp.float32),
        pltpu.VMEM((B, H), jnp.float32),
        pltpu.VMEM((B, H), jnp.float32),
    ],
)


def kernel(x, edge_index, batch, W1, b1, W2, b2, Wl1, bl1, Wl2, bl2):
    ei = edge_index.astype(jnp.int32)
    src = ei[0]
    dst = ei[1]
    pad = EPAD - E
    src_p = jnp.concatenate([src, jnp.zeros((pad,), jnp.int32)])
    dst_p = jnp.concatenate([dst, jnp.full((pad,), N, jnp.int32)])
    src_w = src_p.reshape(NT * NCHT, K)
    dst_w = dst_p.reshape(NT * NCHT, K)
    batch_row = batch.astype(jnp.int32).reshape(1, N)

    degp = _deg(dst_w)
    y1s, dinv = _tc_a(degp, x, W1)
    pp1 = _edge_agg(y1s, src_w, dst_w)
    y2s, g0m, g0x = _tc_b(pp1, y1s, dinv, b1.reshape(1, H), batch_row, W2)
    pp2 = _edge_agg(y2s, src_w, dst_w)
    out = _tc_d(pp2, y2s, dinv, b2.reshape(1, H), batch_row, g0m, g0x,
                Wl1, bl1.reshape(1, 2 * H), Wl2, bl2.reshape(1, C))
    return out
